# baseline (speedup 1.0000x reference)
import jax
import jax.numpy as jnp
from jax.experimental import pallas as pl

D = 256
NX = NY = NZ = 32
NYZ = NY * NZ
DBLK = 64


def _body(xt_ref, yt_ref, zt_ref, out_ref):
    xt = xt_ref[...]
    yt = yt_ref[...]
    zt = zt_ref[...]
    yz = yt[:, :, None] + zt[:, None, :]
    yz = yz.reshape(DBLK, NYZ)
    for x in range(NX):
        out_ref[:, x * NYZ:(x + 1) * NYZ] = xt[:, x][:, None] + yz


def kernel(features, x_embed, y_embed, z_embed):
    b = features.shape[0]
    xt = x_embed[:NX].T
    yt = y_embed[:NY].T
    zt = z_embed[:NZ].T
    grid = (b, D // DBLK)
    out = pl.pallas_call(
        _body,
        grid=grid,
        in_specs=[
            pl.BlockSpec((DBLK, NX), lambda bi, di: (di, 0)),
            pl.BlockSpec((DBLK, NY), lambda bi, di: (di, 0)),
            pl.BlockSpec((DBLK, NZ), lambda bi, di: (di, 0)),
        ],
        out_specs=pl.BlockSpec((DBLK, NX * NYZ), lambda bi, di: (bi * (D // DBLK) + di, 0)),
        out_shape=jax.ShapeDtypeStruct((b * D, NX * NYZ), jnp.float32),
    )(xt, yt, zt)
    return out.reshape(b, D, NX, NY, NZ)


# speedup vs baseline: 2.2704x; 2.2704x over previous
import jax
import jax.numpy as jnp
from jax.experimental import pallas as pl

D = 256
NX = NY = NZ = 32
NYZ = NY * NZ
DBLK = 32


def _body(xt_ref, yt_ref, zt_ref, out_ref):
    xt = xt_ref[...]
    yt = yt_ref[...]
    zt = zt_ref[...]
    yz = yt[:, :, None] + zt[:, None, :]
    yz = yz.reshape(DBLK, 1, NYZ)
    pos = xt[:, :, None] + yz
    out_ref[...] = pos[None]


def kernel(features, x_embed, y_embed, z_embed):
    b = features.shape[0]
    xt = x_embed[:NX].T
    yt = y_embed[:NY].T
    zt = z_embed[:NZ].T
    grid = (b, D // DBLK)
    out = pl.pallas_call(
        _body,
        grid=grid,
        in_specs=[
            pl.BlockSpec((DBLK, NX), lambda bi, di: (di, 0)),
            pl.BlockSpec((DBLK, NY), lambda bi, di: (di, 0)),
            pl.BlockSpec((DBLK, NZ), lambda bi, di: (di, 0)),
        ],
        out_specs=pl.BlockSpec((1, DBLK, NX, NYZ), lambda bi, di: (bi, di, 0, 0)),
        out_shape=jax.ShapeDtypeStruct((b, D, NX, NYZ), jnp.float32),
    )(xt, yt, zt)
    return out.reshape(b, D, NX, NY, NZ)


# speedup vs baseline: 2.4152x; 1.0637x over previous
import jax
import jax.numpy as jnp
from jax.experimental import pallas as pl
from jax.experimental.pallas import tpu as pltpu

D = 256
NX = NY = NZ = 32
NYZ = NY * NZ
DBLK = 32
NSTEP = D // DBLK
NBUF = 2
B = 4


def _body(xt_ref, yt_ref, zt_ref, out_ref, scratch, sems):
    i = pl.program_id(0)
    slot = jax.lax.rem(i, NBUF)

    xt = xt_ref[...]
    yt = yt_ref[...]
    zt = zt_ref[...]
    yz = (yt[:, :, None] + zt[:, None, :]).reshape(DBLK, 1, NYZ)
    pos = xt[:, :, None] + yz

    for k in range(NBUF):
        @pl.when(slot == k)
        def _():
            @pl.when(i >= NBUF)
            def _():
                for bb in range(B):
                    pltpu.make_async_copy(
                        scratch.at[k], out_ref.at[bb, pl.ds((i - NBUF) * DBLK, DBLK)],
                        sems.at[k, bb]).wait()

            scratch[k] = pos

            for bb in range(B):
                pltpu.make_async_copy(
                    scratch.at[k], out_ref.at[bb, pl.ds(i * DBLK, DBLK)],
                    sems.at[k, bb]).start()

    @pl.when(i == NSTEP - 1)
    def _():
        for k in range(NBUF):
            step = i - ((i - k) % NBUF)
            for bb in range(B):
                pltpu.make_async_copy(
                    scratch.at[k], out_ref.at[bb, pl.ds(step * DBLK, DBLK)],
                    sems.at[k, bb]).wait()


def kernel(features, x_embed, y_embed, z_embed):
    b = features.shape[0]
    xt = x_embed[:NX].T
    yt = y_embed[:NY].T
    zt = z_embed[:NZ].T
    out = pl.pallas_call(
        _body,
        grid=(NSTEP,),
        in_specs=[
            pl.BlockSpec((DBLK, NX), lambda i: (i, 0)),
            pl.BlockSpec((DBLK, NY), lambda i: (i, 0)),
            pl.BlockSpec((DBLK, NZ), lambda i: (i, 0)),
        ],
        out_specs=pl.BlockSpec(memory_space=pl.ANY),
        out_shape=jax.ShapeDtypeStruct((b, D, NX, NYZ), jnp.float32),
        scratch_shapes=[
            pltpu.VMEM((NBUF, DBLK, NX, NYZ), jnp.float32),
            pltpu.SemaphoreType.DMA((NBUF, B)),
        ],
    )(xt, yt, zt)
    return out.reshape(b, D, NX, NY, NZ)
